# baseline (speedup 1.0000x reference)
import jax
import jax.numpy as jnp
from jax import lax
from jax.experimental import pallas as pl
from jax.experimental.pallas import tpu as pltpu
from jax.experimental.pallas import tpu_sc as plsc

N = 10000
E = 320000
D = 128

NC = 2
NS = 16
NW = NC * NS

G = 128
KC = 8
CPT = 10
NG_TILE = KC * CPT
E_PAD = NW * CPT * KC * G
NCHUNKS = E_PAD // (KC * G)
RSLOTS = 2

ROWS_A = 632
ROWS_B = N - (NS - 1) * ROWS_A


def _linear_body(x_ref, wt_ref, b_ref, o_ref):
    o_ref[...] = (
        jnp.dot(x_ref[...], wt_ref[...], preferred_element_type=jnp.float32)
        + b_ref[...]
    )


def _linear(x, wt, b2d):
    blk = 2000
    return pl.pallas_call(
        _linear_body,
        grid=(N // blk,),
        in_specs=[
            pl.BlockSpec((blk, D), lambda i: (i, 0)),
            pl.BlockSpec((D, D), lambda i: (0, 0)),
            pl.BlockSpec((1, D), lambda i: (0, 0)),
        ],
        out_specs=pl.BlockSpec((blk, D), lambda i: (i, 0)),
        out_shape=jax.ShapeDtypeStruct((N, D), jnp.float32),
    )(x, wt, b2d)


def _combine_body(p_ref, o_ref):
    o_ref[...] = jnp.maximum(p_ref[0] + p_ref[1], 0.0)


def _combine(partials):
    blk = 2000
    return pl.pallas_call(
        _combine_body,
        grid=(N // blk,),
        in_specs=[pl.BlockSpec((NC, blk, D), lambda i: (0, i, 0))],
        out_specs=pl.BlockSpec((blk, D), lambda i: (i, 0)),
        out_shape=jax.ShapeDtypeStruct((N, D), jnp.float32),
    )(partials)


def _spmm_body(h_hbm, ed_hbm, w_hbm, zeros_hbm, out_hbm,
               ed_buf, w_buf, rows, acc, sem_st, sem_w, sem_g, sem_s):
    c = lax.axis_index("c")
    s = lax.axis_index("s")
    wid = c * NS + s
    chunk0 = wid * CPT

    row0 = s * ROWS_A

    @pl.when(s < NS - 1)
    def _():
        pltpu.sync_copy(zeros_hbm.at[pl.ds(row0, ROWS_A)],
                        acc.at[pl.ds(row0, ROWS_A)])

    @pl.when(s == NS - 1)
    def _():
        pltpu.sync_copy(zeros_hbm.at[pl.ds(row0, ROWS_B)],
                        acc.at[pl.ds(row0, ROWS_B)])

    plsc.subcore_barrier()

    def stage(t, p):
        pltpu.async_copy(ed_hbm.at[chunk0 + t], ed_buf.at[p], sem_st.at[p])
        pltpu.async_copy(w_hbm.at[pl.ds((chunk0 + t) * (KC * G), KC * G)],
                         w_buf.at[p], sem_w.at[p])

    def wait_stage(t, p):
        pltpu.make_async_copy(ed_hbm.at[chunk0 + t], ed_buf.at[p],
                              sem_st.at[p]).wait()
        pltpu.make_async_copy(w_hbm.at[pl.ds((chunk0 + t) * (KC * G), KC * G)],
                              w_buf.at[p], sem_w.at[p]).wait()

    def gather(p, k, r):
        pltpu.async_copy(h_hbm.at[ed_buf.at[p, 0, k]], rows.at[r],
                         sem_g.at[r])

    def wait_gather(p, k, r):
        pltpu.make_async_copy(h_hbm.at[ed_buf.at[p, 0, k]], rows.at[r],
                              sem_g.at[r]).wait()

    def scatter(p, k, r):
        pltpu.async_copy(rows.at[r], acc.at[ed_buf.at[p, 1, k]],
                         sem_s.at[r], add=True)

    def wait_scatter(p, k, r):
        pltpu.make_async_copy(rows.at[r], acc.at[ed_buf.at[p, 1, k]],
                              sem_s.at[r]).wait()

    stage(0, 0)
    wait_stage(0, 0)
    stage(1, 1)
    gather(0, 0, 0)

    @pl.loop(0, NG_TILE)
    def _group(g):
        t = g // KC
        k = g % KC
        r = g % RSLOTS
        p = t % 2
        gn = g + 1
        tn = gn // KC
        kn = gn % KC
        rn = gn % RSLOTS
        pn = tn % 2

        @pl.when(jnp.logical_and(k == 0, jnp.logical_and(t >= 1, t + 1 < CPT)))
        def _():
            stage(t + 1, (t + 1) % 2)

        @pl.when(jnp.logical_and(k == KC - 1, t + 1 < CPT))
        def _():
            wait_stage(t + 1, (t + 1) % 2)

        @pl.when(g >= RSLOTS - 1)
        def _():
            gm = g - (RSLOTS - 1)
            wait_scatter((gm // KC) % 2, gm % KC, gm % RSLOTS)

        @pl.when(gn < NG_TILE)
        def _():
            gather(pn, kn, rn)

        wait_gather(p, k, r)

        @pl.loop(0, G // 16)
        def _edge16(blk16):
            wv16 = w_buf[p, pl.ds(k * G + blk16 * 16, 16)]
            for i in range(16):
                w = wv16[i]
                e = blk16 * 16 + i
                for jj in range(D // 16):
                    sl = pl.ds(jj * 16, 16)
                    rows[r, e, sl] = rows[r, e, sl] * w

        scatter(p, k, r)

    for gm in range(NG_TILE - (RSLOTS - 1), NG_TILE):
        wait_scatter((gm // KC) % 2, gm % KC, gm % RSLOTS)

    plsc.subcore_barrier()

    @pl.when(s < NS - 1)
    def _():
        pltpu.sync_copy(acc.at[pl.ds(row0, ROWS_A)],
                        out_hbm.at[c, pl.ds(row0, ROWS_A)])

    @pl.when(s == NS - 1)
    def _():
        pltpu.sync_copy(acc.at[pl.ds(row0, ROWS_B)],
                        out_hbm.at[c, pl.ds(row0, ROWS_B)])


def _spmm(h, ed, w, zeros):
    mesh = plsc.VectorSubcoreMesh(core_axis_name="c", subcore_axis_name="s")
    kern = pl.kernel(
        _spmm_body,
        out_type=jax.ShapeDtypeStruct((NC, N, D), jnp.float32),
        mesh=mesh,
        scratch_types=[
            pltpu.VMEM((2, 2, KC, G), jnp.int32),
            pltpu.VMEM((2, KC * G), jnp.float32),
            pltpu.VMEM((RSLOTS, G, D), jnp.float32),
            pltpu.VMEM_SHARED((N, D), jnp.float32),
            pltpu.SemaphoreType.DMA((2,)),
            pltpu.SemaphoreType.DMA((2,)),
            pltpu.SemaphoreType.DMA((RSLOTS,)),
            pltpu.SemaphoreType.DMA((RSLOTS,)),
        ],
    )
    return kern(h, ed, w, zeros)


def kernel(x, edge_index, edge_weight, W, b):
    wt = W.T
    b2d = b.reshape(1, D)
    h = _linear(x, wt, b2d)

    pad = E_PAD - E
    src = jnp.concatenate([edge_index[1], jnp.zeros((pad,), jnp.int32)])
    dst = jnp.concatenate([edge_index[0], jnp.zeros((pad,), jnp.int32)])
    w_pad = jnp.concatenate([edge_weight, jnp.zeros((pad,), jnp.float32)])
    ed = jnp.stack(
        [src.reshape(NCHUNKS, KC * G),
         dst.reshape(NCHUNKS, KC * G)], axis=1
    ).reshape(NCHUNKS, 2, KC, G)

    zeros = jnp.zeros((N, D), dtype=jnp.float32)
    partials = _spmm(h, ed, w_pad, zeros)
    return _combine(partials)


# speedup vs baseline: 1.0003x; 1.0003x over previous
import jax
import jax.numpy as jnp
from jax import lax
from jax.experimental import pallas as pl
from jax.experimental.pallas import tpu as pltpu
from jax.experimental.pallas import tpu_sc as plsc

N = 10000
E = 320000
D = 128

NC = 2
NS = 16
NW = NC * NS

G = 128
KC = 8
CPT = 10
NG_TILE = KC * CPT
E_PAD = NW * CPT * KC * G
NCHUNKS = E_PAD // (KC * G)
RSLOTS = 2

ROWS_A = 632
ROWS_B = N - (NS - 1) * ROWS_A


def _linear_body(x_ref, wt_ref, b_ref, o_ref):
    o_ref[...] = (
        jnp.dot(x_ref[...], wt_ref[...], preferred_element_type=jnp.float32)
        + b_ref[...]
    )


def _linear(x, wt, b2d):
    blk = 2000
    return pl.pallas_call(
        _linear_body,
        grid=(N // blk,),
        in_specs=[
            pl.BlockSpec((blk, D), lambda i: (i, 0)),
            pl.BlockSpec((D, D), lambda i: (0, 0)),
            pl.BlockSpec((1, D), lambda i: (0, 0)),
        ],
        out_specs=pl.BlockSpec((blk, D), lambda i: (i, 0)),
        out_shape=jax.ShapeDtypeStruct((N, D), jnp.float32),
    )(x, wt, b2d)


def _combine_body(p_ref, o_ref):
    o_ref[...] = jnp.maximum(p_ref[0] + p_ref[1], 0.0)


def _combine(partials):
    blk = 2000
    return pl.pallas_call(
        _combine_body,
        grid=(N // blk,),
        in_specs=[pl.BlockSpec((NC, blk, D), lambda i: (0, i, 0))],
        out_specs=pl.BlockSpec((blk, D), lambda i: (i, 0)),
        out_shape=jax.ShapeDtypeStruct((N, D), jnp.float32),
    )(partials)


def _spmm_body(h_hbm, ed_hbm, w_hbm, zeros_hbm, out_hbm,
               ed_buf, w_buf, rows, acc, sem_st, sem_w, sem_g, sem_s):
    c = lax.axis_index("c")
    s = lax.axis_index("s")
    wid = c * NS + s
    chunk0 = wid * CPT

    row0 = s * ROWS_A

    @pl.when(s < NS - 1)
    def _():
        pltpu.sync_copy(zeros_hbm.at[pl.ds(row0, ROWS_A)],
                        acc.at[pl.ds(row0, ROWS_A)])

    @pl.when(s == NS - 1)
    def _():
        pltpu.sync_copy(zeros_hbm.at[pl.ds(row0, ROWS_B)],
                        acc.at[pl.ds(row0, ROWS_B)])

    plsc.subcore_barrier()

    def stage(t, p):
        pltpu.async_copy(ed_hbm.at[chunk0 + t], ed_buf.at[p], sem_st.at[p])
        pltpu.async_copy(w_hbm.at[pl.ds((chunk0 + t) * (KC * G), KC * G)],
                         w_buf.at[p], sem_w.at[p])

    def wait_stage(t, p):
        pltpu.make_async_copy(ed_hbm.at[chunk0 + t], ed_buf.at[p],
                              sem_st.at[p]).wait()
        pltpu.make_async_copy(w_hbm.at[pl.ds((chunk0 + t) * (KC * G), KC * G)],
                              w_buf.at[p], sem_w.at[p]).wait()

    def gather(p, k, r):
        pltpu.async_copy(h_hbm.at[ed_buf.at[p, 0, k]], rows.at[r],
                         sem_g.at[r])

    def wait_gather(p, k, r):
        pltpu.make_async_copy(h_hbm.at[ed_buf.at[p, 0, k]], rows.at[r],
                              sem_g.at[r]).wait()

    def scatter(p, k, r):
        pltpu.async_copy(rows.at[r], acc.at[ed_buf.at[p, 1, k]],
                         sem_s.at[r], add=True)

    def wait_scatter(p, k, r):
        pltpu.make_async_copy(rows.at[r], acc.at[ed_buf.at[p, 1, k]],
                              sem_s.at[r]).wait()

    stage(0, 0)
    wait_stage(0, 0)
    stage(1, 1)
    gather(0, 0, 0)

    @pl.loop(0, NG_TILE)
    def _group(g):
        t = g // KC
        k = g % KC
        r = g % RSLOTS
        p = t % 2
        gn = g + 1
        tn = gn // KC
        kn = gn % KC
        rn = gn % RSLOTS
        pn = tn % 2

        @pl.when(jnp.logical_and(k == 0, jnp.logical_and(t >= 1, t + 1 < CPT)))
        def _():
            stage(t + 1, (t + 1) % 2)

        @pl.when(jnp.logical_and(k == KC - 1, t + 1 < CPT))
        def _():
            wait_stage(t + 1, (t + 1) % 2)

        @pl.when(g >= RSLOTS - 1)
        def _():
            gm = g - (RSLOTS - 1)
            wait_scatter((gm // KC) % 2, gm % KC, gm % RSLOTS)

        @pl.when(gn < NG_TILE)
        def _():
            gather(pn, kn, rn)

        wait_gather(p, k, r)

        rows_r = rows.at[r]
        w_pk = w_buf.at[p, pl.ds(k * G, G)]

        @pl.loop(0, G // 16)
        def _edge16(blk16):
            wv16 = w_pk[pl.ds(blk16 * 16, 16)]
            for i in range(16):
                w = wv16[i]
                e = blk16 * 16 + i
                for jj in range(D // 16):
                    sl = pl.ds(jj * 16, 16)
                    rows_r[e, sl] = rows_r[e, sl] * w

        scatter(p, k, r)

    for gm in range(NG_TILE - (RSLOTS - 1), NG_TILE):
        wait_scatter((gm // KC) % 2, gm % KC, gm % RSLOTS)

    plsc.subcore_barrier()

    @pl.when(s < NS - 1)
    def _():
        pltpu.sync_copy(acc.at[pl.ds(row0, ROWS_A)],
                        out_hbm.at[c, pl.ds(row0, ROWS_A)])

    @pl.when(s == NS - 1)
    def _():
        pltpu.sync_copy(acc.at[pl.ds(row0, ROWS_B)],
                        out_hbm.at[c, pl.ds(row0, ROWS_B)])


def _spmm(h, ed, w, zeros):
    mesh = plsc.VectorSubcoreMesh(core_axis_name="c", subcore_axis_name="s")
    kern = pl.kernel(
        _spmm_body,
        out_type=jax.ShapeDtypeStruct((NC, N, D), jnp.float32),
        mesh=mesh,
        scratch_types=[
            pltpu.VMEM((2, 2, KC, G), jnp.int32),
            pltpu.VMEM((2, KC * G), jnp.float32),
            pltpu.VMEM((RSLOTS, G, D), jnp.float32),
            pltpu.VMEM_SHARED((N, D), jnp.float32),
            pltpu.SemaphoreType.DMA((2,)),
            pltpu.SemaphoreType.DMA((2,)),
            pltpu.SemaphoreType.DMA((RSLOTS,)),
            pltpu.SemaphoreType.DMA((RSLOTS,)),
        ],
    )
    return kern(h, ed, w, zeros)


def kernel(x, edge_index, edge_weight, W, b):
    wt = W.T
    b2d = b.reshape(1, D)
    h = _linear(x, wt, b2d)

    pad = E_PAD - E
    src = jnp.concatenate([edge_index[1], jnp.zeros((pad,), jnp.int32)])
    dst = jnp.concatenate([edge_index[0], jnp.zeros((pad,), jnp.int32)])
    w_pad = jnp.concatenate([edge_weight, jnp.zeros((pad,), jnp.float32)])
    ed = jnp.stack(
        [src.reshape(NCHUNKS, KC * G),
         dst.reshape(NCHUNKS, KC * G)], axis=1
    ).reshape(NCHUNKS, 2, KC, G)

    zeros = jnp.zeros((N, D), dtype=jnp.float32)
    partials = _spmm(h, ed, w_pad, zeros)
    return _combine(partials)
